# baseline (speedup 1.0000x reference)
import functools

import jax
import jax.numpy as jnp
from jax import lax
from jax.experimental import pallas as pl
from jax.experimental.pallas import tpu as pltpu
from jax.experimental.pallas import tpu_sc as plsc

B = 16384
F = 26
V = 100000
D = 32

VCHUNK = 4096
NVBLK = 25
VPAD = NVBLK * VCHUNK

_INFO = plsc.get_sparse_core_info()
NC = _INFO.num_cores
NS = _INFO.num_subcores
NW = NC * NS
L = _INFO.num_lanes

TOTAL = B * F
R = TOTAL // NW
IDXW = 128
G = 13
C = G * IDXW
NCH = R // C


def _detile_body(tabt_ref, out_ref):
    a = tabt_ref[0]
    eye = jnp.eye(D, dtype=jnp.float32)
    eye3 = jnp.concatenate([eye, eye, eye], axis=0)
    for s in range(VCHUNK // 512):
        blk = a[:, s * 512:(s + 1) * 512]
        hi = blk.astype(jnp.bfloat16).astype(jnp.float32)
        r1 = blk - hi
        mid = r1.astype(jnp.bfloat16).astype(jnp.float32)
        lo = r1 - mid
        stacked = jnp.concatenate([hi, mid, lo], axis=0)
        dn = (((0,), (0,)), ((), ()))
        bt = lax.dot_general(
            stacked, eye3, dn, preferred_element_type=jnp.float32
        )
        out_ref[s * 128:(s + 1) * 128, :] = jnp.concatenate(
            [bt[t * 128:(t + 1) * 128, :] for t in range(4)], axis=1
        )


_detile = pl.pallas_call(
    _detile_body,
    grid=(F, NVBLK),
    in_specs=[pl.BlockSpec((1, D, VCHUNK), lambda f, j: (f, 0, j))],
    out_specs=pl.BlockSpec((VCHUNK // 4, 128), lambda f, j: (f * NVBLK + j, 0)),
    out_shape=jax.ShapeDtypeStruct((F * NVBLK * (VCHUNK // 4), 128), jnp.float32),
)


def _embed_body(x_hbm, tab_hbm, out_hbm, idx0, idx1, off_v, rows0, rows1,
                gsem0, gsem1, wsem0, wsem1):
    wid = lax.axis_index("s") * NC + lax.axis_index("c")
    base = wid * R

    idx_b = (idx0, idx1)
    rows_b = (rows0, rows1)
    gsem_b = (gsem0, gsem1)
    wsem_b = (wsem0, wsem1)

    def off_body(k, carry):
        lanes = k * L + lax.iota(jnp.int32, L)
        off_v[pl.ds(k * L, L)] = lax.rem(lanes, F) * VPAD
        return carry

    lax.fori_loop(0, C // L, off_body, 0)

    def stage(c):
        b = c % 2
        pltpu.sync_copy(x_hbm.at[pl.ds(base + c * C, C)], idx_b[b])

        def add_body(k, carry):
            sl = pl.ds(k * L, L)
            v = idx_b[b][sl]
            row = (
                off_v[sl]
                + (v & -512)
                + ((v & 127) << 2)
                + ((v >> 7) & 3)
            )
            idx_b[b][sl] = row
            return carry

        lax.fori_loop(0, C // L, add_body, 0)

    def fire(c):
        b = c % 2
        return [
            pltpu.async_copy(
                tab_hbm.at[idx_b[b].at[pl.ds(g * IDXW, IDXW)]],
                rows_b[b].at[pl.ds(g * IDXW, IDXW)],
                gsem_b[b],
            )
            for g in range(G)
        ]

    def writeback(c):
        b = c % 2
        return pltpu.async_copy(
            rows_b[b], out_hbm.at[pl.ds(base + c * C, C)], wsem_b[b]
        )

    wb = [None] * NCH
    stage(0)
    gathers = fire(0)
    for c in range(1, NCH):
        if c >= 2:
            wb[c - 2].wait()
        stage(c)
        prev_gathers = gathers
        gathers = fire(c)
        for cp in prev_gathers:
            cp.wait()
        wb[c - 1] = writeback(c - 1)
    wb[NCH - 2].wait()
    for cp in gathers:
        cp.wait()
    writeback(NCH - 1).wait()


@functools.partial(
    pl.kernel,
    out_type=jax.ShapeDtypeStruct((TOTAL, D), jnp.float32),
    mesh=plsc.VectorSubcoreMesh(core_axis_name="c", subcore_axis_name="s"),
    compiler_params=pltpu.CompilerParams(use_tc_tiling_on_sc=False),
    scratch_types=[
        pltpu.VMEM((C,), jnp.int32),
        pltpu.VMEM((C,), jnp.int32),
        pltpu.VMEM((C,), jnp.int32),
        pltpu.VMEM((C, D), jnp.float32),
        pltpu.VMEM((C, D), jnp.float32),
        pltpu.SemaphoreType.DMA,
        pltpu.SemaphoreType.DMA,
        pltpu.SemaphoreType.DMA,
        pltpu.SemaphoreType.DMA,
    ],
)
def _embed(x_hbm, tab_hbm, out_hbm, idx0, idx1, off_v, rows0, rows1,
           gsem0, gsem1, wsem0, wsem1):
    _embed_body(x_hbm, tab_hbm, out_hbm, idx0, idx1, off_v, rows0, rows1,
                gsem0, gsem1, wsem0, wsem1)


def kernel(X, tables):
    tab_t = jnp.transpose(tables, (0, 2, 1))
    dense128 = _detile(tab_t)
    tab_flat = dense128.reshape(F * VPAD, D)
    x_flat = X.reshape(TOTAL)
    out = _embed(x_flat, tab_flat)
    return out.reshape(B, 1, F * D)


# speedup vs baseline: 1.1818x; 1.1818x over previous
import functools

import jax
import jax.numpy as jnp
from jax import lax
from jax.experimental import pallas as pl
from jax.experimental.pallas import tpu as pltpu
from jax.experimental.pallas import tpu_sc as plsc

B = 16384
F = 26
V = 100000
D = 32

VCHUNK = 51200
NVBLK = 2
VPAD = NVBLK * VCHUNK

_INFO = plsc.get_sparse_core_info()
NC = _INFO.num_cores
NS = _INFO.num_subcores
NW = NC * NS
L = _INFO.num_lanes

TOTAL = B * F
R = TOTAL // NW
IDXW = 128
G = 13
C = G * IDXW
NCH = R // C


def _detile_body(tabt_ref, out_ref):
    a = tabt_ref[0]
    eye = jnp.eye(D, dtype=jnp.float32)
    eye3 = jnp.concatenate([eye, eye, eye], axis=0)
    dn = (((0,), (0,)), ((), ()))
    for m in range(VCHUNK // 2048):
        blk = a[:, m * 2048:(m + 1) * 2048]
        hi = blk.astype(jnp.bfloat16).astype(jnp.float32)
        r1 = blk - hi
        mid = r1.astype(jnp.bfloat16).astype(jnp.float32)
        lo = r1 - mid
        stacked = jnp.concatenate([hi, mid, lo], axis=0)
        bt = lax.dot_general(
            stacked, eye3, dn, preferred_element_type=jnp.float32
        )
        for u in range(4):
            s = m * 4 + u
            out_ref[s * 128:(s + 1) * 128, :] = jnp.concatenate(
                [
                    bt[u * 512 + t * 128:u * 512 + (t + 1) * 128, :]
                    for t in range(4)
                ],
                axis=1,
            )


_detile = pl.pallas_call(
    _detile_body,
    grid=(F, NVBLK),
    in_specs=[pl.BlockSpec((1, D, VCHUNK), lambda f, j: (f, 0, j))],
    out_specs=pl.BlockSpec((VCHUNK // 4, 128), lambda f, j: (f * NVBLK + j, 0)),
    out_shape=jax.ShapeDtypeStruct((F * NVBLK * (VCHUNK // 4), 128), jnp.float32),
)


def _embed_body(x_hbm, tab_hbm, out_hbm, idx0, idx1, off_v, rows0, rows1,
                gsem0, gsem1, wsem0, wsem1):
    wid = lax.axis_index("s") * NC + lax.axis_index("c")
    base = wid * R

    idx_b = (idx0, idx1)
    rows_b = (rows0, rows1)
    gsem_b = (gsem0, gsem1)
    wsem_b = (wsem0, wsem1)

    def off_body(k, carry):
        lanes = k * L + lax.iota(jnp.int32, L)
        off_v[pl.ds(k * L, L)] = lax.rem(lanes, F) * VPAD
        return carry

    lax.fori_loop(0, C // L, off_body, 0)

    def stage(c):
        b = c % 2
        pltpu.sync_copy(x_hbm.at[pl.ds(base + c * C, C)], idx_b[b])

        def add_body(k, carry):
            sl = pl.ds(k * L, L)
            v = idx_b[b][sl]
            row = (
                off_v[sl]
                + (v & -512)
                + ((v & 127) << 2)
                + ((v >> 7) & 3)
            )
            idx_b[b][sl] = row
            return carry

        lax.fori_loop(0, C // L, add_body, 0)

    def fire(c):
        b = c % 2
        return [
            pltpu.async_copy(
                tab_hbm.at[idx_b[b].at[pl.ds(g * IDXW, IDXW)]],
                rows_b[b].at[pl.ds(g * IDXW, IDXW)],
                gsem_b[b],
            )
            for g in range(G)
        ]

    def writeback(c):
        b = c % 2
        return pltpu.async_copy(
            rows_b[b], out_hbm.at[pl.ds(base + c * C, C)], wsem_b[b]
        )

    wb = [None] * NCH
    stage(0)
    gathers = fire(0)
    for c in range(1, NCH):
        if c >= 2:
            wb[c - 2].wait()
        stage(c)
        prev_gathers = gathers
        gathers = fire(c)
        for cp in prev_gathers:
            cp.wait()
        wb[c - 1] = writeback(c - 1)
    wb[NCH - 2].wait()
    for cp in gathers:
        cp.wait()
    writeback(NCH - 1).wait()


@functools.partial(
    pl.kernel,
    out_type=jax.ShapeDtypeStruct((TOTAL, D), jnp.float32),
    mesh=plsc.VectorSubcoreMesh(core_axis_name="c", subcore_axis_name="s"),
    compiler_params=pltpu.CompilerParams(use_tc_tiling_on_sc=False),
    scratch_types=[
        pltpu.VMEM((C,), jnp.int32),
        pltpu.VMEM((C,), jnp.int32),
        pltpu.VMEM((C,), jnp.int32),
        pltpu.VMEM((C, D), jnp.float32),
        pltpu.VMEM((C, D), jnp.float32),
        pltpu.SemaphoreType.DMA,
        pltpu.SemaphoreType.DMA,
        pltpu.SemaphoreType.DMA,
        pltpu.SemaphoreType.DMA,
    ],
)
def _embed(x_hbm, tab_hbm, out_hbm, idx0, idx1, off_v, rows0, rows1,
           gsem0, gsem1, wsem0, wsem1):
    _embed_body(x_hbm, tab_hbm, out_hbm, idx0, idx1, off_v, rows0, rows1,
                gsem0, gsem1, wsem0, wsem1)


def kernel(X, tables):
    tab_t = jnp.transpose(tables, (0, 2, 1))
    dense128 = _detile(tab_t)
    tab_flat = dense128.reshape(F * VPAD, D)
    x_flat = X.reshape(TOTAL)
    out = _embed(x_flat, tab_flat)
    return out.reshape(B, 1, F * D)


# speedup vs baseline: 1.2933x; 1.0943x over previous
import functools

import jax
import jax.numpy as jnp
from jax import lax
from jax.experimental import pallas as pl
from jax.experimental.pallas import tpu as pltpu
from jax.experimental.pallas import tpu_sc as plsc

B = 16384
F = 26
V = 100000
D = 32

VCHUNK = 51200
NVBLK = 2
VPAD = NVBLK * VCHUNK

_INFO = plsc.get_sparse_core_info()
NC = _INFO.num_cores
NS = _INFO.num_subcores
NW = NC * NS
L = _INFO.num_lanes

TOTAL = B * F
R = TOTAL // NW
IDXW = 128
G = 13
C = G * IDXW
NCH = R // C


def _split3(blk):
    hi = blk.astype(jnp.bfloat16).astype(jnp.float32)
    r1 = blk - hi
    mid = r1.astype(jnp.bfloat16).astype(jnp.float32)
    lo = r1 - mid
    return [hi, mid, lo]


def _detile_body(tabt_ref, out_ref):
    a = tabt_ref[0]
    eye = jnp.eye(D, dtype=jnp.float32)
    eye3 = jnp.concatenate([eye, eye, eye], axis=0)
    zero3 = jnp.zeros((3 * D, D), jnp.float32)
    rhs = jnp.concatenate(
        [
            jnp.concatenate([eye3, zero3], axis=1),
            jnp.concatenate([zero3, eye3], axis=1),
        ],
        axis=0,
    )
    dn = (((0,), (0,)), ((), ()))
    for m in range(VCHUNK // 1024):
        g0 = a[:, m * 1024:m * 1024 + 512]
        g1 = a[:, m * 1024 + 512:(m + 1) * 1024]
        stacked = jnp.concatenate(_split3(g0) + _split3(g1), axis=0)
        bt = lax.dot_general(
            stacked, rhs, dn, preferred_element_type=jnp.float32
        )
        for u in range(2):
            s = m * 2 + u
            out_ref[s * 128:(s + 1) * 128, :] = jnp.concatenate(
                [
                    bt[t * 128:(t + 1) * 128, u * D:(u + 1) * D]
                    for t in range(4)
                ],
                axis=1,
            )


_detile = pl.pallas_call(
    _detile_body,
    grid=(F, NVBLK),
    in_specs=[pl.BlockSpec((1, D, VCHUNK), lambda f, j: (f, 0, j))],
    out_specs=pl.BlockSpec((VCHUNK // 4, 128), lambda f, j: (f * NVBLK + j, 0)),
    out_shape=jax.ShapeDtypeStruct((F * NVBLK * (VCHUNK // 4), 128), jnp.float32),
)


def _embed_body(x_hbm, tab_hbm, out_hbm, idx0, idx1, off_v, rows0, rows1,
                gsem0, gsem1, wsem0, wsem1):
    wid = lax.axis_index("s") * NC + lax.axis_index("c")
    base = wid * R

    idx_b = (idx0, idx1)
    rows_b = (rows0, rows1)
    gsem_b = (gsem0, gsem1)
    wsem_b = (wsem0, wsem1)

    def off_body(k, carry):
        lanes = k * L + lax.iota(jnp.int32, L)
        off_v[pl.ds(k * L, L)] = lax.rem(lanes, F) * VPAD
        return carry

    lax.fori_loop(0, C // L, off_body, 0)

    def stage(c):
        b = c % 2
        pltpu.sync_copy(x_hbm.at[pl.ds(base + c * C, C)], idx_b[b])

        def add_body(k, carry):
            sl = pl.ds(k * L, L)
            v = idx_b[b][sl]
            row = (
                off_v[sl]
                + (v & -512)
                + ((v & 127) << 2)
                + ((v >> 7) & 3)
            )
            idx_b[b][sl] = row
            return carry

        lax.fori_loop(0, C // L, add_body, 0)

    def fire(c):
        b = c % 2
        return [
            pltpu.async_copy(
                tab_hbm.at[idx_b[b].at[pl.ds(g * IDXW, IDXW)]],
                rows_b[b].at[pl.ds(g * IDXW, IDXW)],
                gsem_b[b],
            )
            for g in range(G)
        ]

    def writeback(c):
        b = c % 2
        return pltpu.async_copy(
            rows_b[b], out_hbm.at[pl.ds(base + c * C, C)], wsem_b[b]
        )

    wb = [None] * NCH
    stage(0)
    gathers = fire(0)
    for c in range(1, NCH):
        if c >= 2:
            wb[c - 2].wait()
        stage(c)
        prev_gathers = gathers
        gathers = fire(c)
        for cp in prev_gathers:
            cp.wait()
        wb[c - 1] = writeback(c - 1)
    wb[NCH - 2].wait()
    for cp in gathers:
        cp.wait()
    writeback(NCH - 1).wait()


@functools.partial(
    pl.kernel,
    out_type=jax.ShapeDtypeStruct((TOTAL, D), jnp.float32),
    mesh=plsc.VectorSubcoreMesh(core_axis_name="c", subcore_axis_name="s"),
    compiler_params=pltpu.CompilerParams(use_tc_tiling_on_sc=False),
    scratch_types=[
        pltpu.VMEM((C,), jnp.int32),
        pltpu.VMEM((C,), jnp.int32),
        pltpu.VMEM((C,), jnp.int32),
        pltpu.VMEM((C, D), jnp.float32),
        pltpu.VMEM((C, D), jnp.float32),
        pltpu.SemaphoreType.DMA,
        pltpu.SemaphoreType.DMA,
        pltpu.SemaphoreType.DMA,
        pltpu.SemaphoreType.DMA,
    ],
)
def _embed(x_hbm, tab_hbm, out_hbm, idx0, idx1, off_v, rows0, rows1,
           gsem0, gsem1, wsem0, wsem1):
    _embed_body(x_hbm, tab_hbm, out_hbm, idx0, idx1, off_v, rows0, rows1,
                gsem0, gsem1, wsem0, wsem1)


def kernel(X, tables):
    tab_t = jnp.transpose(tables, (0, 2, 1))
    dense128 = _detile(tab_t)
    tab_flat = dense128.reshape(F * VPAD, D)
    x_flat = X.reshape(TOTAL)
    out = _embed(x_flat, tab_flat)
    return out.reshape(B, 1, F * D)
